# baseline (speedup 1.0000x reference)
import functools

import jax
import jax.numpy as jnp
from jax import lax
from jax.experimental import pallas as pl
from jax.experimental.pallas import tpu as pltpu
from jax.experimental.pallas import tpu_sc as plsc

M = 1000000
D = 64
B = 16384

_info = plsc.get_sparse_core_info()
_NC = _info.num_cores
_NS = _info.num_subcores
_NW = _NC * _NS
_B_PER_W = B // _NW


def _make_gather():
    mesh = plsc.VectorSubcoreMesh(core_axis_name="c", subcore_axis_name="s")

    @functools.partial(
        pl.kernel,
        mesh=mesh,
        out_type=jax.ShapeDtypeStruct((B, 2 * D), jnp.float32),
        scratch_types=[
            pltpu.VMEM((_B_PER_W,), jnp.int32),
            pltpu.VMEM((_B_PER_W, 2 * D), jnp.float32),
            pltpu.SemaphoreType.DMA,
        ],
    )
    def gather(table_hbm, pair_hbm, out_hbm, idx_v, rows_v, sem):
        wid = lax.axis_index("s") * _NC + lax.axis_index("c")
        base = wid * _B_PER_W
        pltpu.sync_copy(pair_hbm.at[pl.ds(base, _B_PER_W)], idx_v)
        pltpu.async_copy(table_hbm.at[idx_v], rows_v, sem).wait()
        pltpu.sync_copy(rows_v, out_hbm.at[pl.ds(base, _B_PER_W)])

    return gather


_gather = _make_gather()


@jax.jit
def kernel(old_mems, new_mems, indices):
    del old_mems
    idx = indices.astype(jnp.int32)
    pairs = _gather(new_mems.reshape(M // 2, 2 * D), idx >> 1)
    odd = (idx & 1).astype(jnp.bool_)
    return jnp.where(odd[:, None], pairs[:, D:], pairs[:, :D])


# speedup vs baseline: 1.0427x; 1.0427x over previous
import functools

import jax
import jax.numpy as jnp
from jax import lax
from jax.experimental import pallas as pl
from jax.experimental.pallas import tpu as pltpu
from jax.experimental.pallas import tpu_sc as plsc

M = 1000000
D = 64
B = 16384

_info = plsc.get_sparse_core_info()
_NC = _info.num_cores
_NS = _info.num_subcores
_NW = _NC * _NS
_B_PER_W = B // _NW
_L = 16
_NGROUP = _B_PER_W // _L


def _make_gather():
    mesh = plsc.VectorSubcoreMesh(core_axis_name="c", subcore_axis_name="s")

    @functools.partial(
        pl.kernel,
        mesh=mesh,
        out_type=jax.ShapeDtypeStruct((B, D), jnp.float32),
        scratch_types=[
            pltpu.VMEM((_B_PER_W,), jnp.int32),
            pltpu.SemaphoreType.DMA,
        ],
    )
    def gather(table_hbm, idx_hbm, out_hbm, idx_v, gsem):
        wid = lax.axis_index("s") * _NC + lax.axis_index("c")
        base = wid * _B_PER_W
        pltpu.sync_copy(idx_hbm.at[wid], idx_v)

        def group(g, carry):
            v16 = idx_v[pl.ds(g * _L, _L)]
            for l in range(_L):
                s = v16[l]
                pltpu.async_copy(
                    table_hbm.at[s], out_hbm.at[base + g * _L + l], gsem
                )
            return carry

        lax.fori_loop(0, _NGROUP, group, 0)
        pltpu.make_async_copy(
            table_hbm.at[pl.ds(0, _B_PER_W)],
            out_hbm.at[pl.ds(base, _B_PER_W)],
            gsem,
        ).wait()

    return gather


_gather = _make_gather()


@jax.jit
def kernel(old_mems, new_mems, indices):
    del old_mems
    idx = indices.astype(jnp.int32).reshape(_NW, _B_PER_W)
    return _gather(new_mems, idx)
